# baseline (speedup 1.0000x reference)
import functools

import jax
import jax.numpy as jnp
from jax import lax
from jax.experimental import pallas as pl
from jax.experimental.pallas import tpu as pltpu
from jax.experimental.pallas import tpu_sc as plsc

T = 2048
D = 768
E = 8
F = 1024
SF = 2048
BT = 256
NA = 2 * T
NB = NA // BT + E
NROWS = NB * BT
NW = 32
RPT = NROWS // NW
TPT = T // NW


def _silu(x):
    return x * jax.nn.sigmoid(x)


def _router_kernel(x_ref, gw_ref, gb_ref, ids_ref, rank_ref, w0_ref, w1_ref,
                   cnt_ref, carry_ref):
    i = pl.program_id(0)

    @pl.when(i == 0)
    def _():
        carry_ref[...] = jnp.zeros((1, 16), jnp.float32)

    xb = x_ref[...]
    xbb = xb.astype(jnp.bfloat16)
    logits = jax.lax.dot_general(
        xbb, gw_ref[...].astype(jnp.bfloat16), (((1,), (1,)), ((), ())),
        preferred_element_type=jnp.float32)
    m = jnp.max(logits, axis=-1, keepdims=True)
    ex = jnp.exp(logits - m)
    probs = ex / jnp.sum(ex, axis=-1, keepdims=True)
    sel = probs + gb_ref[...]

    eids = jax.lax.broadcasted_iota(jnp.int32, (BT, E), 1)
    i0 = jnp.argmax(sel, axis=-1)[:, None]
    sel2 = jnp.where(eids == i0, -jnp.inf, sel)
    i1 = jnp.argmax(sel2, axis=-1)[:, None]
    oh0 = (eids == i0).astype(jnp.float32)
    oh1 = (eids == i1).astype(jnp.float32)
    p0 = jnp.sum(oh0 * probs, axis=-1, keepdims=True)
    p1 = jnp.sum(oh1 * probs, axis=-1, keepdims=True)
    denom = p0 + p1 + 1e-9

    r = jax.lax.broadcasted_iota(jnp.int32, (BT, BT), 0)
    c = jax.lax.broadcasted_iota(jnp.int32, (BT, BT), 1)
    ltri = (r > c).astype(jnp.bfloat16)
    oh_tot = oh0 + oh1
    cexc = jnp.dot(ltri, oh_tot.astype(jnp.bfloat16),
                   preferred_element_type=jnp.float32)
    carry = carry_ref[...][:, :E]
    rank_all = cexc + carry
    rank0 = jnp.sum(oh0 * rank_all, axis=-1, keepdims=True)
    rank1 = jnp.sum(oh1 * rank_all, axis=-1, keepdims=True)

    new_carry = carry + jnp.sum(oh_tot, axis=0, keepdims=True)
    carry_ref[...] = jnp.concatenate(
        [new_carry, jnp.zeros((1, 16 - E), jnp.float32)], axis=1)
    cnt_ref[...] = carry_ref[...].astype(jnp.int32)

    ids_ref[...] = jnp.concatenate([i0, i1], axis=1)
    rank_ref[...] = jnp.concatenate([rank0, rank1], axis=1).astype(jnp.int32)
    w0_ref[...] = jnp.broadcast_to(p0 / denom, (BT, 16))
    w1_ref[...] = jnp.broadcast_to(p1 / denom, (BT, 16))


def _run_router(x, gate_w, gb):
    return pl.pallas_call(
        _router_kernel,
        grid=(T // BT,),
        in_specs=[
            pl.BlockSpec((BT, D), lambda i: (i, 0)),
            pl.BlockSpec((E, D), lambda i: (0, 0)),
            pl.BlockSpec((1, E), lambda i: (0, 0)),
        ],
        out_specs=[
            pl.BlockSpec((BT, 2), lambda i: (i, 0)),
            pl.BlockSpec((BT, 2), lambda i: (i, 0)),
            pl.BlockSpec((BT, 16), lambda i: (i, 0)),
            pl.BlockSpec((BT, 16), lambda i: (i, 0)),
            pl.BlockSpec((1, 16), lambda i: (0, 0)),
        ],
        out_shape=[
            jax.ShapeDtypeStruct((T, 2), jnp.int32),
            jax.ShapeDtypeStruct((T, 2), jnp.int32),
            jax.ShapeDtypeStruct((T, 16), jnp.float32),
            jax.ShapeDtypeStruct((T, 16), jnp.float32),
            jax.ShapeDtypeStruct((1, 16), jnp.int32),
        ],
        scratch_shapes=[pltpu.VMEM((1, 16), jnp.float32)],
    )(x, gate_w, gb)


def _shared_kernel(x_ref, sguw_ref, sdw_ref, out_ref, sgb_ref, sdb_ref):
    @pl.when(pl.program_id(0) == 0)
    def _():
        sgb_ref[...] = sguw_ref[...].astype(jnp.bfloat16)
        sdb_ref[...] = sdw_ref[...].astype(jnp.bfloat16)

    sh = jnp.dot(x_ref[...].astype(jnp.bfloat16), sgb_ref[...],
                 preferred_element_type=jnp.float32)
    sa = (_silu(sh[:, :SF]) * sh[:, SF:]).astype(jnp.bfloat16)
    out_ref[...] = jnp.dot(sa, sdb_ref[...],
                           preferred_element_type=jnp.float32)


def _run_shared(x, sguw, sdw):
    return pl.pallas_call(
        _shared_kernel,
        grid=(T // BT,),
        in_specs=[
            pl.BlockSpec((BT, D), lambda i: (i, 0)),
            pl.BlockSpec((D, 2 * SF), lambda i: (0, 0)),
            pl.BlockSpec((SF, D), lambda i: (0, 0)),
        ],
        out_specs=pl.BlockSpec((BT, D), lambda i: (i, 0)),
        out_shape=jax.ShapeDtypeStruct((T, D), jnp.float32),
        scratch_shapes=[
            pltpu.VMEM((D, 2 * SF), jnp.bfloat16),
            pltpu.VMEM((SF, D), jnp.bfloat16),
        ],
    )(x, sguw, sdw)


def _sc_dispatch(x_hbm, e0_hbm, e1_hbm, r0_hbm, r1_hbm, cnt_hbm,
                 xg_hbm, pos0_hbm, pos1_hbm, bexp_hbm, bpar_hbm, meta_hbm,
                 ebuf0, ebuf1, rbuf0, rbuf1, cnt_v, pos2buf,
                 bexp_v, bpar_v, meta_v, xbuf, sem0, sem1):
    wid = lax.axis_index("s") * 2 + lax.axis_index("c")
    t0 = wid * TPT

    pltpu.sync_copy(cnt_hbm, cnt_v)
    i16 = lax.iota(jnp.int32, 16)
    cntv = cnt_v[...]
    pcs = [((cntv[e] + (BT - 1)) >> 8) << 8 for e in range(E)]
    offs = [jnp.int32(0)]
    for e in range(E - 1):
        offs.append(offs[e] + pcs[e])
    nrows = offs[E - 1] + pcs[E - 1]

    @pl.when(wid == 0)
    def _():
        for h in range(2):
            base = (i16 + h * 16) * BT
            bexp = jnp.full((16,), -1, jnp.int32)
            scnt = jnp.full((16,), -1, jnp.int32)
            for e in range(E):
                bexp = bexp + jnp.where(base >= offs[e], 1, 0)
                ne = jnp.where(pcs[e] > 0, 1, 0)
                scnt = scnt + jnp.where(base >= offs[e], ne, 0)
            bexp = jnp.where(base < nrows, bexp, E - 1)
            bexp_v[pl.ds(h * 16, 16)] = bexp
            bpar_v[pl.ds(h * 16, 16)] = scnt & 1
        meta_v[...] = jnp.where(i16 == 0, nrows, 0)
        pltpu.sync_copy(bexp_v, bexp_hbm)
        pltpu.sync_copy(bpar_v, bpar_hbm)
        pltpu.sync_copy(meta_v, meta_hbm)

    sl_tok = pl.ds(t0, TPT)
    cpx = pltpu.async_copy(x_hbm.at[sl_tok], xbuf, sem0)
    pltpu.sync_copy(e0_hbm.at[sl_tok], ebuf0)
    pltpu.sync_copy(e1_hbm.at[sl_tok], ebuf1)
    pltpu.sync_copy(r0_hbm.at[sl_tok], rbuf0)
    pltpu.sync_copy(r1_hbm.at[sl_tok], rbuf1)

    for k, (eb, rb) in enumerate(((ebuf0, rbuf0), (ebuf1, rbuf1))):
        for j in range(TPT // 16):
            sl = pl.ds(j * 16, 16)
            e_vec = eb[sl]
            off_sel = jnp.zeros((16,), jnp.int32)
            for e in range(E):
                off_sel = off_sel + jnp.where(e_vec == e, offs[e], 0)
            pos2buf[k, sl] = off_sel + rb[sl]
    pltpu.sync_copy(pos2buf.at[0], pos0_hbm.at[sl_tok])
    pltpu.sync_copy(pos2buf.at[1], pos1_hbm.at[sl_tok])

    cpx.wait()
    cp0 = pltpu.async_copy(xbuf, xg_hbm.at[pos2buf.at[0]], sem0)
    cp1 = pltpu.async_copy(xbuf, xg_hbm.at[pos2buf.at[1]], sem1)
    cp0.wait()
    cp1.wait()


def _run_dispatch(x, e0, e1, r0, r1, cnt16):
    mesh = plsc.VectorSubcoreMesh(core_axis_name="c", subcore_axis_name="s")
    fn = functools.partial(
        pl.kernel,
        mesh=mesh,
        out_type=[
            jax.ShapeDtypeStruct((NROWS, D), jnp.float32),
            jax.ShapeDtypeStruct((T,), jnp.int32),
            jax.ShapeDtypeStruct((T,), jnp.int32),
            jax.ShapeDtypeStruct((32,), jnp.int32),
            jax.ShapeDtypeStruct((32,), jnp.int32),
            jax.ShapeDtypeStruct((16,), jnp.int32),
        ],
        scratch_types=[
            pltpu.VMEM((TPT,), jnp.int32),
            pltpu.VMEM((TPT,), jnp.int32),
            pltpu.VMEM((TPT,), jnp.int32),
            pltpu.VMEM((TPT,), jnp.int32),
            pltpu.VMEM((16,), jnp.int32),
            pltpu.VMEM((2, TPT), jnp.int32),
            pltpu.VMEM((32,), jnp.int32),
            pltpu.VMEM((32,), jnp.int32),
            pltpu.VMEM((16,), jnp.int32),
            pltpu.VMEM((TPT, D), jnp.float32),
            pltpu.SemaphoreType.DMA,
            pltpu.SemaphoreType.DMA,
        ],
    )(_sc_dispatch)
    return fn(x, e0, e1, r0, r1, cnt16)


def _grouped_kernel(bexp_ref, bpar_ref, meta_ref, xg_ref, guw_hbm, dw_hbm,
                    yg_ref, wg_ref, wd_ref, wgb_ref, wdb_ref, semg, semd):
    i = pl.program_id(0)
    e_i = bexp_ref[i]
    p_i = bpar_ref[i]
    nrows = meta_ref[0]
    used = i * BT < nrows
    first_use = jnp.logical_and(
        used, jnp.logical_or(i == 0, bexp_ref[jnp.maximum(i - 1, 0)] != e_i))

    @pl.when(i == 0)
    def _():
        pltpu.make_async_copy(guw_hbm.at[e_i], wg_ref.at[p_i],
                              semg.at[p_i]).start()
        pltpu.make_async_copy(dw_hbm.at[e_i], wd_ref.at[p_i],
                              semd.at[p_i]).start()

    if_next = jnp.logical_and(i + 1 < NB,
                              jnp.logical_and(bexp_ref[jnp.minimum(i + 1, NB - 1)] != e_i,
                                              (i + 1) * BT < nrows))

    @pl.when(if_next)
    def _():
        e_n = bexp_ref[jnp.minimum(i + 1, NB - 1)]
        p_n = bpar_ref[jnp.minimum(i + 1, NB - 1)]
        pltpu.make_async_copy(guw_hbm.at[e_n], wg_ref.at[p_n],
                              semg.at[p_n]).start()
        pltpu.make_async_copy(dw_hbm.at[e_n], wd_ref.at[p_n],
                              semd.at[p_n]).start()

    @pl.when(first_use)
    def _():
        pltpu.make_async_copy(guw_hbm.at[e_i], wg_ref.at[p_i],
                              semg.at[p_i]).wait()
        pltpu.make_async_copy(dw_hbm.at[e_i], wd_ref.at[p_i],
                              semd.at[p_i]).wait()
        wgb_ref[p_i] = wg_ref[p_i].astype(jnp.bfloat16)
        wdb_ref[p_i] = wd_ref[p_i].astype(jnp.bfloat16)

    @pl.when(used)
    def _():
        h = jnp.dot(xg_ref[...].astype(jnp.bfloat16), wgb_ref[p_i],
                    preferred_element_type=jnp.float32)
        a = (_silu(h[:, :F]) * h[:, F:]).astype(jnp.bfloat16)
        yg_ref[...] = jnp.dot(a, wdb_ref[p_i],
                              preferred_element_type=jnp.float32)


def _run_grouped(bexp, bpar, meta, xg, guw, dw):
    grid_spec = pltpu.PrefetchScalarGridSpec(
        num_scalar_prefetch=3,
        grid=(NB,),
        in_specs=[
            pl.BlockSpec((BT, D), lambda i, b, p, m: (i, 0)),
            pl.BlockSpec(memory_space=pl.ANY),
            pl.BlockSpec(memory_space=pl.ANY),
        ],
        out_specs=pl.BlockSpec((BT, D), lambda i, b, p, m: (i, 0)),
        scratch_shapes=[
            pltpu.VMEM((2, D, 2 * F), jnp.float32),
            pltpu.VMEM((2, F, D), jnp.float32),
            pltpu.VMEM((2, D, 2 * F), jnp.bfloat16),
            pltpu.VMEM((2, F, D), jnp.bfloat16),
            pltpu.SemaphoreType.DMA((2,)),
            pltpu.SemaphoreType.DMA((2,)),
        ],
    )
    return pl.pallas_call(
        _grouped_kernel,
        grid_spec=grid_spec,
        out_shape=jax.ShapeDtypeStruct((NROWS, D), jnp.float32),
    )(bexp, bpar, meta, xg, guw, dw)


_CH = 16


def _sc_combine(pos0_hbm, pos1_hbm, w0_hbm, w1_hbm, yg_hbm, sh_hbm, out_hbm,
                p0_v, p1_v, w0buf, w1buf, ygbuf0, ygbuf1, shbuf, outbuf,
                sem0, sem1, semsh):
    wid = lax.axis_index("s") * 2 + lax.axis_index("c")
    base = wid * TPT
    slt = pl.ds(base, TPT)
    pltpu.sync_copy(pos0_hbm.at[slt], p0_v)
    pltpu.sync_copy(pos1_hbm.at[slt], p1_v)
    pltpu.sync_copy(w0_hbm.at[slt], w0buf)
    pltpu.sync_copy(w1_hbm.at[slt], w1buf)
    nch = TPT // _CH

    def issue(c):
        b = c % 2
        sl16 = pl.ds(c * _CH, _CH)
        return [
            pltpu.async_copy(yg_hbm.at[p0_v.at[sl16]], ygbuf0.at[b], sem0),
            pltpu.async_copy(yg_hbm.at[p1_v.at[sl16]], ygbuf1.at[b], sem1),
            pltpu.async_copy(sh_hbm.at[pl.ds(base + c * _CH, _CH)],
                             shbuf.at[b], semsh),
        ]

    cps = issue(0)
    for c in range(nch):
        for cp in cps:
            cp.wait()
        if c + 1 < nch:
            cps = issue(c + 1)
        b = c % 2

        def body(j, _):
            w0v = w0buf[c * _CH + j, :]
            w1v = w1buf[c * _CH + j, :]

            def inner(d, _):
                dsl = pl.ds(d * 16, 16)
                outbuf[j, dsl] = (shbuf[b, j, dsl] + w0v * ygbuf0[b, j, dsl]
                                  + w1v * ygbuf1[b, j, dsl])
                return 0
            lax.fori_loop(0, D // 16, inner, 0)
            return 0

        lax.fori_loop(0, _CH, body, 0)
        pltpu.sync_copy(outbuf, out_hbm.at[pl.ds(base + c * _CH, _CH)])


def _run_combine(pos0, pos1, w0rep, w1rep, yg, shared):
    mesh = plsc.VectorSubcoreMesh(core_axis_name="c", subcore_axis_name="s")
    fn = functools.partial(
        pl.kernel,
        mesh=mesh,
        out_type=jax.ShapeDtypeStruct((T, D), jnp.float32),
        scratch_types=[
            pltpu.VMEM((TPT,), jnp.int32),
            pltpu.VMEM((TPT,), jnp.int32),
            pltpu.VMEM((TPT, 16), jnp.float32),
            pltpu.VMEM((TPT, 16), jnp.float32),
            pltpu.VMEM((2, _CH, D), jnp.float32),
            pltpu.VMEM((2, _CH, D), jnp.float32),
            pltpu.VMEM((2, _CH, D), jnp.float32),
            pltpu.VMEM((_CH, D), jnp.float32),
            pltpu.SemaphoreType.DMA,
            pltpu.SemaphoreType.DMA,
            pltpu.SemaphoreType.DMA,
        ],
    )(_sc_combine)
    return fn(pos0, pos1, w0rep, w1rep, yg, shared)


@jax.jit
def kernel(x, gate_w, gate_bias, gate_up_w, down_w, shared_gate_up_w,
           shared_down_w):
    gb = gate_bias.reshape(1, E)

    ids, rank, w0rep, w1rep, cnt = _run_router(x, gate_w, gb)
    xg, pos0, pos1, bexp, bpar, meta = _run_dispatch(
        x, ids[:, 0], ids[:, 1], rank[:, 0], rank[:, 1], cnt.reshape(16))
    shared = _run_shared(x, shared_gate_up_w, shared_down_w)
    yg = _run_grouped(bexp, bpar, meta, xg, gate_up_w, down_w)
    out = _run_combine(pos0, pos1, w0rep, w1rep, yg, shared)
    return out


# speedup vs baseline: 1.1386x; 1.1386x over previous
import functools

import jax
import jax.numpy as jnp
from jax import lax
from jax.experimental import pallas as pl
from jax.experimental.pallas import tpu as pltpu
from jax.experimental.pallas import tpu_sc as plsc

T = 2048
D = 768
E = 8
F = 1024
SF = 2048
BT = 256
NA = 2 * T
NB = NA // BT + E
NROWS = NB * BT
NW = 32
RPT = NROWS // NW
TPT = T // NW


def _silu(x):
    return x * jax.nn.sigmoid(x)


def _router_kernel(x_ref, gw_ref, gb_ref, ids_ref, rank_ref, w0_ref, w1_ref,
                   cnt_ref, carry_ref):
    i = pl.program_id(0)

    @pl.when(i == 0)
    def _():
        carry_ref[...] = jnp.zeros((1, 16), jnp.float32)

    xb = x_ref[...]
    xbb = xb.astype(jnp.bfloat16)
    logits = jax.lax.dot_general(
        xbb, gw_ref[...].astype(jnp.bfloat16), (((1,), (1,)), ((), ())),
        preferred_element_type=jnp.float32)
    m = jnp.max(logits, axis=-1, keepdims=True)
    ex = jnp.exp(logits - m)
    probs = ex / jnp.sum(ex, axis=-1, keepdims=True)
    sel = probs + gb_ref[...]

    eids = jax.lax.broadcasted_iota(jnp.int32, (BT, E), 1)
    i0 = jnp.argmax(sel, axis=-1)[:, None]
    sel2 = jnp.where(eids == i0, -jnp.inf, sel)
    i1 = jnp.argmax(sel2, axis=-1)[:, None]
    oh0 = (eids == i0).astype(jnp.float32)
    oh1 = (eids == i1).astype(jnp.float32)
    p0 = jnp.sum(oh0 * probs, axis=-1, keepdims=True)
    p1 = jnp.sum(oh1 * probs, axis=-1, keepdims=True)
    denom = p0 + p1 + 1e-9

    r = jax.lax.broadcasted_iota(jnp.int32, (BT, BT), 0)
    c = jax.lax.broadcasted_iota(jnp.int32, (BT, BT), 1)
    ltri = (r > c).astype(jnp.bfloat16)
    oh_tot = oh0 + oh1
    cexc = jnp.dot(ltri, oh_tot.astype(jnp.bfloat16),
                   preferred_element_type=jnp.float32)
    carry = carry_ref[...][:, :E]
    rank_all = cexc + carry
    rank0 = jnp.sum(oh0 * rank_all, axis=-1, keepdims=True)
    rank1 = jnp.sum(oh1 * rank_all, axis=-1, keepdims=True)

    new_carry = carry + jnp.sum(oh_tot, axis=0, keepdims=True)
    carry_ref[...] = jnp.concatenate(
        [new_carry, jnp.zeros((1, 16 - E), jnp.float32)], axis=1)
    cnt_ref[...] = carry_ref[...].astype(jnp.int32)

    ids_ref[...] = jnp.concatenate([i0, i1], axis=1)
    rank_ref[...] = jnp.concatenate([rank0, rank1], axis=1).astype(jnp.int32)
    w0_ref[...] = jnp.broadcast_to(p0 / denom, (BT, 16))
    w1_ref[...] = jnp.broadcast_to(p1 / denom, (BT, 16))


def _run_router(x, gate_w, gb):
    return pl.pallas_call(
        _router_kernel,
        grid=(T // BT,),
        in_specs=[
            pl.BlockSpec((BT, D), lambda i: (i, 0)),
            pl.BlockSpec((E, D), lambda i: (0, 0)),
            pl.BlockSpec((1, E), lambda i: (0, 0)),
        ],
        out_specs=[
            pl.BlockSpec((BT, 2), lambda i: (i, 0)),
            pl.BlockSpec((BT, 2), lambda i: (i, 0)),
            pl.BlockSpec((BT, 16), lambda i: (i, 0)),
            pl.BlockSpec((BT, 16), lambda i: (i, 0)),
            pl.BlockSpec((1, 16), lambda i: (0, 0)),
        ],
        out_shape=[
            jax.ShapeDtypeStruct((T, 2), jnp.int32),
            jax.ShapeDtypeStruct((T, 2), jnp.int32),
            jax.ShapeDtypeStruct((T, 16), jnp.float32),
            jax.ShapeDtypeStruct((T, 16), jnp.float32),
            jax.ShapeDtypeStruct((1, 16), jnp.int32),
        ],
        scratch_shapes=[pltpu.VMEM((1, 16), jnp.float32)],
    )(x, gate_w, gb)


def _shared_kernel(x_ref, sguw_ref, sdw_ref, out_ref):
    sh = jnp.dot(x_ref[...], sguw_ref[...], preferred_element_type=jnp.float32)
    sa = _silu(sh[:, :SF]) * sh[:, SF:]
    out_ref[...] = jnp.dot(sa, sdw_ref[...], preferred_element_type=jnp.float32)


def _run_shared(x, sguw, sdw):
    return pl.pallas_call(
        _shared_kernel,
        grid=(T // BT,),
        in_specs=[
            pl.BlockSpec((BT, D), lambda i: (i, 0)),
            pl.BlockSpec((D, 2 * SF), lambda i: (0, 0)),
            pl.BlockSpec((SF, D), lambda i: (0, 0)),
        ],
        out_specs=pl.BlockSpec((BT, D), lambda i: (i, 0)),
        out_shape=jax.ShapeDtypeStruct((T, D), jnp.float32),
    )(x, sguw, sdw)


def _sc_dispatch(x_hbm, e0_hbm, e1_hbm, r0_hbm, r1_hbm, cnt_hbm,
                 xg_hbm, pos0_hbm, pos1_hbm, bexp_hbm, bpar_hbm, meta_hbm,
                 ebuf0, ebuf1, rbuf0, rbuf1, cnt_v, pos2buf,
                 bexp_v, bpar_v, meta_v, xbuf, sem0, sem1):
    wid = lax.axis_index("s") * 2 + lax.axis_index("c")
    t0 = wid * TPT

    pltpu.sync_copy(cnt_hbm, cnt_v)
    i16 = lax.iota(jnp.int32, 16)
    cntv = cnt_v[...]
    pcs = [((cntv[e] + (BT - 1)) >> 8) << 8 for e in range(E)]
    offs = [jnp.int32(0)]
    for e in range(E - 1):
        offs.append(offs[e] + pcs[e])
    nrows = offs[E - 1] + pcs[E - 1]

    @pl.when(wid == 0)
    def _():
        for h in range(2):
            base = (i16 + h * 16) * BT
            bexp = jnp.full((16,), -1, jnp.int32)
            scnt = jnp.full((16,), -1, jnp.int32)
            for e in range(E):
                bexp = bexp + jnp.where(base >= offs[e], 1, 0)
                ne = jnp.where(pcs[e] > 0, 1, 0)
                scnt = scnt + jnp.where(base >= offs[e], ne, 0)
            bexp = jnp.where(base < nrows, bexp, E - 1)
            bexp_v[pl.ds(h * 16, 16)] = bexp
            nxt = jnp.full((16,), -1, jnp.int32)
            for e in range(E - 1, 0, -1):
                ne = jnp.where(pcs[e] > 0, 1, 0)
                cand = jnp.where(base < offs[e], ne, 0)
                nxt = cand * e + (1 - cand) * nxt
            bpar_v[pl.ds(h * 16, 16)] = (scnt & 1) + 2 * (nxt + 1)
        meta_v[...] = jnp.where(i16 == 0, nrows, 0)
        pltpu.sync_copy(bexp_v, bexp_hbm)
        pltpu.sync_copy(bpar_v, bpar_hbm)
        pltpu.sync_copy(meta_v, meta_hbm)

    sl_tok = pl.ds(t0, TPT)
    cpx = pltpu.async_copy(x_hbm.at[sl_tok], xbuf, sem0)
    pltpu.sync_copy(e0_hbm.at[sl_tok], ebuf0)
    pltpu.sync_copy(e1_hbm.at[sl_tok], ebuf1)
    pltpu.sync_copy(r0_hbm.at[sl_tok], rbuf0)
    pltpu.sync_copy(r1_hbm.at[sl_tok], rbuf1)

    for k, (eb, rb) in enumerate(((ebuf0, rbuf0), (ebuf1, rbuf1))):
        for j in range(TPT // 16):
            sl = pl.ds(j * 16, 16)
            e_vec = eb[sl]
            off_sel = jnp.zeros((16,), jnp.int32)
            for e in range(E):
                off_sel = off_sel + jnp.where(e_vec == e, offs[e], 0)
            pos2buf[k, sl] = off_sel + rb[sl]
    pltpu.sync_copy(pos2buf.at[0], pos0_hbm.at[sl_tok])
    pltpu.sync_copy(pos2buf.at[1], pos1_hbm.at[sl_tok])

    cpx.wait()
    cp0 = pltpu.async_copy(xbuf, xg_hbm.at[pos2buf.at[0]], sem0)
    cp1 = pltpu.async_copy(xbuf, xg_hbm.at[pos2buf.at[1]], sem1)
    cp0.wait()
    cp1.wait()


def _run_dispatch(x, e0, e1, r0, r1, cnt16):
    mesh = plsc.VectorSubcoreMesh(core_axis_name="c", subcore_axis_name="s")
    fn = functools.partial(
        pl.kernel,
        mesh=mesh,
        out_type=[
            jax.ShapeDtypeStruct((NROWS, D), jnp.float32),
            jax.ShapeDtypeStruct((T,), jnp.int32),
            jax.ShapeDtypeStruct((T,), jnp.int32),
            jax.ShapeDtypeStruct((32,), jnp.int32),
            jax.ShapeDtypeStruct((32,), jnp.int32),
            jax.ShapeDtypeStruct((16,), jnp.int32),
        ],
        scratch_types=[
            pltpu.VMEM((TPT,), jnp.int32),
            pltpu.VMEM((TPT,), jnp.int32),
            pltpu.VMEM((TPT,), jnp.int32),
            pltpu.VMEM((TPT,), jnp.int32),
            pltpu.VMEM((16,), jnp.int32),
            pltpu.VMEM((2, TPT), jnp.int32),
            pltpu.VMEM((32,), jnp.int32),
            pltpu.VMEM((32,), jnp.int32),
            pltpu.VMEM((16,), jnp.int32),
            pltpu.VMEM((TPT, D), jnp.float32),
            pltpu.SemaphoreType.DMA,
            pltpu.SemaphoreType.DMA,
        ],
    )(_sc_dispatch)
    return fn(x, e0, e1, r0, r1, cnt16)


def _grouped_kernel(bexp_ref, bpar_ref, meta_ref, xg_ref, guw_hbm, dw_hbm,
                    yg_ref, wg_ref, wd_ref, semg, semd):
    i = pl.program_id(0)
    e_i = bexp_ref[i]
    v_i = bpar_ref[i]
    p_i = v_i & 1
    e_n = (v_i >> 1) - 1
    p_n = 1 - p_i
    nrows = meta_ref[0]
    used = i * BT < nrows
    first_use = jnp.logical_and(
        used, jnp.logical_or(i == 0, bexp_ref[jnp.maximum(i - 1, 0)] != e_i))

    @pl.when(i == 0)
    def _():
        pltpu.make_async_copy(guw_hbm.at[e_i], wg_ref.at[p_i],
                              semg.at[p_i]).start()
        pltpu.make_async_copy(dw_hbm.at[e_i], wd_ref.at[p_i],
                              semd.at[p_i]).start()

    @pl.when(first_use)
    def _():
        pltpu.make_async_copy(guw_hbm.at[e_i], wg_ref.at[p_i],
                              semg.at[p_i]).wait()
        pltpu.make_async_copy(dw_hbm.at[e_i], wd_ref.at[p_i],
                              semd.at[p_i]).wait()

    @pl.when(jnp.logical_and(first_use, e_n >= 0))
    def _():
        en = jnp.maximum(e_n, 0)
        pltpu.make_async_copy(guw_hbm.at[en], wg_ref.at[p_n],
                              semg.at[p_n]).start()
        pltpu.make_async_copy(dw_hbm.at[en], wd_ref.at[p_n],
                              semd.at[p_n]).start()

    @pl.when(used)
    def _():
        h = jnp.dot(xg_ref[...], wg_ref[p_i],
                    preferred_element_type=jnp.float32)
        a = _silu(h[:, :F]) * h[:, F:]
        yg_ref[...] = jnp.dot(a, wd_ref[p_i],
                              preferred_element_type=jnp.float32)


def _run_grouped(bexp, bpar, meta, xg, guw, dw):
    grid_spec = pltpu.PrefetchScalarGridSpec(
        num_scalar_prefetch=3,
        grid=(NB,),
        in_specs=[
            pl.BlockSpec((BT, D), lambda i, b, p, m: (i, 0)),
            pl.BlockSpec(memory_space=pl.ANY),
            pl.BlockSpec(memory_space=pl.ANY),
        ],
        out_specs=pl.BlockSpec((BT, D), lambda i, b, p, m: (i, 0)),
        scratch_shapes=[
            pltpu.VMEM((2, D, 2 * F), jnp.float32),
            pltpu.VMEM((2, F, D), jnp.float32),
            pltpu.SemaphoreType.DMA((2,)),
            pltpu.SemaphoreType.DMA((2,)),
        ],
    )
    return pl.pallas_call(
        _grouped_kernel,
        grid_spec=grid_spec,
        out_shape=jax.ShapeDtypeStruct((NROWS, D), jnp.float32),
    )(bexp, bpar, meta, xg, guw, dw)


_CH = 16


def _sc_combine(pos0_hbm, pos1_hbm, w0_hbm, w1_hbm, yg_hbm, sh_hbm, out_hbm,
                p0_v, p1_v, w0buf, w1buf, ygbuf0, ygbuf1, shbuf, outbuf,
                sem0, sem1, semsh):
    wid = lax.axis_index("s") * 2 + lax.axis_index("c")
    base = wid * TPT
    slt = pl.ds(base, TPT)
    pltpu.sync_copy(pos0_hbm.at[slt], p0_v)
    pltpu.sync_copy(pos1_hbm.at[slt], p1_v)
    pltpu.sync_copy(w0_hbm.at[slt], w0buf)
    pltpu.sync_copy(w1_hbm.at[slt], w1buf)
    nch = TPT // _CH

    def issue(c):
        b = c % 2
        sl16 = pl.ds(c * _CH, _CH)
        return [
            pltpu.async_copy(yg_hbm.at[p0_v.at[sl16]], ygbuf0.at[b], sem0),
            pltpu.async_copy(yg_hbm.at[p1_v.at[sl16]], ygbuf1.at[b], sem1),
            pltpu.async_copy(sh_hbm.at[pl.ds(base + c * _CH, _CH)],
                             shbuf.at[b], semsh),
        ]

    cps = issue(0)
    for c in range(nch):
        for cp in cps:
            cp.wait()
        if c + 1 < nch:
            cps = issue(c + 1)
        b = c % 2

        def body(j, _):
            w0v = w0buf[c * _CH + j, :]
            w1v = w1buf[c * _CH + j, :]
            for d in range(D // 16):
                dsl = pl.ds(d * 16, 16)
                outbuf[j, dsl] = (shbuf[b, j, dsl] + w0v * ygbuf0[b, j, dsl]
                                  + w1v * ygbuf1[b, j, dsl])
            return 0

        lax.fori_loop(0, _CH, body, 0)
        pltpu.sync_copy(outbuf, out_hbm.at[pl.ds(base + c * _CH, _CH)])


def _run_combine(pos0, pos1, w0rep, w1rep, yg, shared):
    mesh = plsc.VectorSubcoreMesh(core_axis_name="c", subcore_axis_name="s")
    fn = functools.partial(
        pl.kernel,
        mesh=mesh,
        out_type=jax.ShapeDtypeStruct((T, D), jnp.float32),
        scratch_types=[
            pltpu.VMEM((TPT,), jnp.int32),
            pltpu.VMEM((TPT,), jnp.int32),
            pltpu.VMEM((TPT, 16), jnp.float32),
            pltpu.VMEM((TPT, 16), jnp.float32),
            pltpu.VMEM((2, _CH, D), jnp.float32),
            pltpu.VMEM((2, _CH, D), jnp.float32),
            pltpu.VMEM((2, _CH, D), jnp.float32),
            pltpu.VMEM((_CH, D), jnp.float32),
            pltpu.SemaphoreType.DMA,
            pltpu.SemaphoreType.DMA,
            pltpu.SemaphoreType.DMA,
        ],
    )(_sc_combine)
    return fn(pos0, pos1, w0rep, w1rep, yg, shared)


@jax.jit
def kernel(x, gate_w, gate_bias, gate_up_w, down_w, shared_gate_up_w,
           shared_down_w):
    gb = gate_bias.reshape(1, E)

    ids, rank, w0rep, w1rep, cnt = _run_router(x, gate_w, gb)
    xg, pos0, pos1, bexp, bpar, meta = _run_dispatch(
        x, ids[:, 0], ids[:, 1], rank[:, 0], rank[:, 1], cnt.reshape(16))
    shared = _run_shared(x, shared_gate_up_w, shared_down_w)
    yg = _run_grouped(bexp, bpar, meta, xg, gate_up_w, down_w)
    out = _run_combine(pos0, pos1, w0rep, w1rep, yg, shared)
    return out
